# baseline (speedup 1.0000x reference)
import jax
import jax.numpy as jnp
from jax.experimental import pallas as pl
from jax.experimental.pallas import tpu as pltpu

_KX = 8
_KE = 32


def _copy_body(x_ref, ea_ref, xo_ref, eo_ref, sems_x, sems_e):
    copies = []
    rx = 10000 // _KX
    for i in range(_KX):
        c = pltpu.make_async_copy(
            x_ref.at[pl.ds(i * rx, rx), :],
            xo_ref.at[pl.ds(i * rx, rx), :],
            sems_x.at[i],
        )
        c.start()
        copies.append(c)
    re = 40000 // _KE
    for i in range(_KE):
        c = pltpu.make_async_copy(
            ea_ref.at[pl.ds(i * re, re), :],
            eo_ref.at[pl.ds(i * re, re), :],
            sems_e.at[i],
        )
        c.start()
        copies.append(c)
    for c in copies:
        c.wait()


def kernel(x, edge_index, edge_attr):
    ea2 = edge_attr.reshape(40000, 128)
    x_out, ea_out = pl.pallas_call(
        _copy_body,
        out_shape=(
            jax.ShapeDtypeStruct((10000, 128), x.dtype),
            jax.ShapeDtypeStruct((40000, 128), edge_attr.dtype),
        ),
        in_specs=[
            pl.BlockSpec(memory_space=pl.ANY),
            pl.BlockSpec(memory_space=pl.ANY),
        ],
        out_specs=(
            pl.BlockSpec(memory_space=pl.ANY),
            pl.BlockSpec(memory_space=pl.ANY),
        ),
        scratch_shapes=[
            pltpu.SemaphoreType.DMA((_KX,)),
            pltpu.SemaphoreType.DMA((_KE,)),
        ],
    )(x, ea2)
    return (x_out, ea_out.reshape(320000, 16))


# speedup vs baseline: 3.5162x; 3.5162x over previous
import jax
import jax.numpy as jnp
from jax.experimental import pallas as pl
from jax.experimental.pallas import tpu as pltpu


def _copy_body(x_ref, ea_ref, xo_ref, eo_ref, xbuf, ebuf, sems):
    in_x = pltpu.make_async_copy(x_ref, xbuf, sems.at[0])
    in_e = pltpu.make_async_copy(ea_ref, ebuf, sems.at[1])
    in_x.start()
    in_e.start()
    in_x.wait()
    out_x = pltpu.make_async_copy(xbuf, xo_ref, sems.at[2])
    out_x.start()
    in_e.wait()
    out_e = pltpu.make_async_copy(ebuf, eo_ref, sems.at[3])
    out_e.start()
    out_x.wait()
    out_e.wait()


def kernel(x, edge_index, edge_attr):
    ea2 = edge_attr.reshape(40000, 128)
    x_out, ea_out = pl.pallas_call(
        _copy_body,
        out_shape=(
            jax.ShapeDtypeStruct((10000, 128), x.dtype),
            jax.ShapeDtypeStruct((40000, 128), edge_attr.dtype),
        ),
        in_specs=[
            pl.BlockSpec(memory_space=pl.ANY),
            pl.BlockSpec(memory_space=pl.ANY),
        ],
        out_specs=(
            pl.BlockSpec(memory_space=pl.ANY),
            pl.BlockSpec(memory_space=pl.ANY),
        ),
        scratch_shapes=[
            pltpu.MemorySpace.VMEM((10000, 128), jnp.float32),
            pltpu.MemorySpace.VMEM((40000, 128), jnp.float32),
            pltpu.SemaphoreType.DMA((4,)),
        ],
    )(x, ea2)
    return (x_out, ea_out.reshape(320000, 16))
